# Initial kernel scaffold; baseline (speedup 1.0000x reference)
import jax
import jax.numpy as jnp
from jax.experimental import pallas as pl


def kernel(x, qb, upg_w1, upg_w2, upg_b2, gate_norm_w, up_proj_w, pb, down_norm_w, pos_emb, db, down_proj_w, res_norm_w, cb, sb):
    raise NotImplementedError("write your pallas kernel here")


# speedup vs baseline: 1.0941x; 1.0941x over previous
import functools

import jax
import jax.numpy as jnp
from jax.experimental import pallas as pl

_D = 1024
_S = 2048
_SI = 512
_NH = 4
_DH = _D // _NH
_LN_EPS = 1e-5
_RMS_EPS = 1e-8
_CH = 256
_PREC = jax.lax.Precision.DEFAULT
_EXACT = jax.lax.Precision.HIGHEST


def _ln_rows(x, w):
    mu = jnp.mean(x, axis=-1, keepdims=True)
    var = jnp.mean((x - mu) ** 2, axis=-1, keepdims=True)
    return (x - mu) / jnp.sqrt(var + _LN_EPS) * w


def _mm_kern(*refs, ln, rms, gelu, res, nw):
    it = iter(refs)
    x = next(it)[...]
    if ln:
        x = _ln_rows(x, next(it)[...])
    if rms:
        x = x / jnp.sqrt(jnp.mean(x * x, axis=-1, keepdims=True) + _RMS_EPS)
    outs = []
    for _ in range(nw):
        w = next(it)[...]
        outs.append(jnp.dot(x, w, preferred_element_type=jnp.float32, precision=_PREC))
    acc = outs[0] if nw == 1 else jnp.concatenate(outs, axis=1)
    if gelu:
        acc = jax.nn.gelu(acc)
    if res:
        acc = acc + next(it)[...]
    next(it)[...] = acc


def _mm(x, ws, ln_w=None, rms_in=False, gelu_out=False, resid=None, mt=256):
    M, K = x.shape
    N = sum(w.shape[1] for w in ws)
    inputs = [x]
    in_specs = [pl.BlockSpec((mt, K), lambda i: (i, 0))]
    if ln_w is not None:
        inputs.append(ln_w.reshape(1, K))
        in_specs.append(pl.BlockSpec((1, K), lambda i: (0, 0)))
    for w in ws:
        inputs.append(w)
        in_specs.append(pl.BlockSpec(w.shape, lambda i: (0, 0)))
    if resid is not None:
        inputs.append(resid)
        in_specs.append(pl.BlockSpec((mt, N), lambda i: (i, 0)))
    kern = functools.partial(_mm_kern, ln=ln_w is not None, rms=rms_in,
                             gelu=gelu_out, res=resid is not None, nw=len(ws))
    return pl.pallas_call(
        kern, grid=(M // mt,),
        in_specs=in_specs,
        out_specs=pl.BlockSpec((mt, N), lambda i: (i, 0)),
        out_shape=jax.ShapeDtypeStruct((M, N), jnp.float32),
    )(*inputs)


def _attn_kern(q_ref, k_ref, v_ref, o_ref, *, causal, sk, tq):
    i = pl.program_id(1)
    q = q_ref[...]
    k = k_ref[...]
    v = v_ref[...]
    s = jax.lax.dot_general(q, k, (((1,), (1,)), ((), ())),
                            preferred_element_type=jnp.float32, precision=_PREC) * (1.0 / 16.0)
    if causal:
        row = jax.lax.broadcasted_iota(jnp.int32, (tq, sk), 0) + i * tq
        col = jax.lax.broadcasted_iota(jnp.int32, (tq, sk), 1)
        s = jnp.where(row >= col, s, -1e30)
    m = jnp.max(s, axis=-1, keepdims=True)
    p = jnp.exp(s - m)
    p = p / jnp.sum(p, axis=-1, keepdims=True)
    o_ref[...] = jnp.dot(p, v, preferred_element_type=jnp.float32, precision=_PREC)


def _attn(qarr, kvarr, causal, k_off, v_off, tq=512):
    sq = qarr.shape[0]
    sk = kvarr.shape[0]
    kern = functools.partial(_attn_kern, causal=causal, sk=sk, tq=tq)
    return pl.pallas_call(
        kern, grid=(_NH, sq // tq),
        in_specs=[
            pl.BlockSpec((tq, _DH), lambda h, i: (i, h)),
            pl.BlockSpec((sk, _DH), lambda h, i, o=k_off: (0, o + h)),
            pl.BlockSpec((sk, _DH), lambda h, i, o=v_off: (0, o + h)),
        ],
        out_specs=pl.BlockSpec((tq, _DH), lambda h, i: (i, h)),
        out_shape=jax.ShapeDtypeStruct((sq, _D), jnp.float32),
    )(qarr, kvarr, kvarr)


def _block(x, p):
    qkv = _mm(x, [p['wq'], p['wk'], p['wv']], ln_w=p['ln1'])
    ctx = _attn(qkv, qkv, False, _NH, 2 * _NH)
    x2 = _mm(ctx[:, :_D], [p['wo']], resid=x)
    h = _mm(x2, [p['w1']], ln_w=p['ln2'], gelu_out=True)
    return _mm(h, [p['w2']], resid=x2)


def _combine(kv, q, p):
    qp = _mm(q, [p['wq']], ln_w=p['lnq'])
    kvp = _mm(kv, [p['wk'], p['wv']], ln_w=p['lnkv'])
    ctx = _attn(qp, kvp, True, 0, _NH)
    x2 = _mm(ctx, [p['wo']], resid=q)
    h = _mm(x2, [p['w1']], ln_w=p['ln2'], gelu_out=True)
    return _mm(h, [p['w2']], resid=x2)


def _route_kern(g_ref, w1_ref, w2_ref, b2_ref, gnc_ref, gnr_ref, upw_ref,
                up_ref, mt_ref):
    g = g_ref[...]
    h1 = jax.nn.gelu(jnp.dot(g, w1_ref[...],
                             preferred_element_type=jnp.float32, precision=_PREC))
    gp = jnp.dot(h1, w2_ref[...],
                 preferred_element_type=jnp.float32, precision=_PREC) + b2_ref[0, 0]
    mu = jnp.mean(gp)
    var = jnp.mean((gp - mu) ** 2)
    inv = 1.0 / jnp.sqrt(var + _LN_EPS)
    gate_c = (gp - mu) * inv * gnc_ref[...]
    parts = []
    for c in range(_S // _CH):
        e = (jax.lax.broadcasted_iota(jnp.int32, (_S, _CH), 0) ==
             jax.lax.broadcasted_iota(jnp.int32, (_S, _CH), 1) + c * _CH)
        parts.append(jax.lax.dot_general(
            gp, e.astype(jnp.float32), (((0,), (0,)), ((), ())),
            preferred_element_type=jnp.float32, precision=_EXACT))
    gp_r = jnp.concatenate(parts, axis=1)
    gate_r = (gp_r - mu) * inv * gnr_ref[...]
    ii = jax.lax.broadcasted_iota(jnp.int32, (_S, _CH), 0)
    rank = jnp.zeros((_S, 1), jnp.float32)
    for c in range(_S // _CH):
        gj = gate_r[:, c * _CH:(c + 1) * _CH]
        jj = jax.lax.broadcasted_iota(jnp.int32, (_S, _CH), 1) + c * _CH
        gt = gj > gate_c
        tie = (gj == gate_c) & (jj < ii)
        rank = rank + jnp.sum((gt | tie).astype(jnp.float32), axis=1,
                              keepdims=True)
    mask = (rank < float(_SI)).astype(jnp.float32)
    spos = jnp.zeros((_S, 1), jnp.float32)
    for c in range(_S // _CH):
        mj = mask[c * _CH:(c + 1) * _CH, :]
        jj = jax.lax.broadcasted_iota(jnp.int32, (_S, _CH), 1) + c * _CH
        cc = (jj < ii).astype(jnp.float32)
        spos = spos + jnp.dot(cc, mj, preferred_element_type=jnp.float32, precision=_EXACT)
    sidx = jax.lax.broadcasted_iota(jnp.int32, (_S, _SI), 1)
    mt = ((spos.astype(jnp.int32) == sidx) &
          (mask > 0.0)).astype(jnp.float32)
    mt_ref[...] = mt
    gsel = jax.lax.dot_general(mt, gate_c, (((0,), (0,)), ((), ())),
                               preferred_element_type=jnp.float32, precision=_EXACT)
    grms = g / jnp.sqrt(jnp.mean(g * g, axis=-1, keepdims=True) + _RMS_EPS)
    sel = jax.lax.dot_general(mt, grms, (((0,), (0,)), ((), ())),
                              preferred_element_type=jnp.float32, precision=_EXACT)
    sel = sel * jax.nn.sigmoid(gsel)
    up_ref[...] = jnp.dot(sel, upw_ref[...],
                          preferred_element_type=jnp.float32, precision=_PREC)


def _route(gathered, upg_w1, upg_w2, upg_b2, gate_norm_w, up_proj_w):
    return pl.pallas_call(
        _route_kern,
        out_shape=(jax.ShapeDtypeStruct((_SI, _D), jnp.float32),
                   jax.ShapeDtypeStruct((_S, _SI), jnp.float32)),
    )(gathered, upg_w1, upg_w2, upg_b2.reshape(1, 1),
      gate_norm_w.reshape(_S, 1), gate_norm_w.reshape(1, _S), up_proj_w)


def _scatter_kern(mt_ref, xd_ref, dnw_ref, pos_ref, o_ref):
    xn = _ln_rows(xd_ref[...], dnw_ref[...])
    o_ref[...] = jnp.dot(mt_ref[...], xn,
                         preferred_element_type=jnp.float32,
                         precision=_EXACT) + pos_ref[...]


def _scatter(mt, up_out, down_norm_w, pos_emb, bt=256):
    return pl.pallas_call(
        _scatter_kern, grid=(_S // bt,),
        in_specs=[
            pl.BlockSpec((bt, _SI), lambda i: (i, 0)),
            pl.BlockSpec((_SI, _D), lambda i: (0, 0)),
            pl.BlockSpec((1, _D), lambda i: (0, 0)),
            pl.BlockSpec((bt, _D), lambda i: (i, 0)),
        ],
        out_specs=pl.BlockSpec((bt, _D), lambda i: (i, 0)),
        out_shape=jax.ShapeDtypeStruct((_S, _D), jnp.float32),
    )(mt, up_out, down_norm_w.reshape(1, _D), pos_emb)


def _addln_kern(x_ref, p_ref, w_ref, o_ref):
    o_ref[...] = _ln_rows(x_ref[...] + p_ref[...], w_ref[...])


def _addln(x, pos, w, bt=256):
    return pl.pallas_call(
        _addln_kern, grid=(_S // bt,),
        in_specs=[
            pl.BlockSpec((bt, _D), lambda i: (i, 0)),
            pl.BlockSpec((bt, _D), lambda i: (i, 0)),
            pl.BlockSpec((1, _D), lambda i: (0, 0)),
        ],
        out_specs=pl.BlockSpec((bt, _D), lambda i: (i, 0)),
        out_shape=jax.ShapeDtypeStruct((_S, _D), jnp.float32),
    )(x, pos, w.reshape(1, _D))


def kernel(x, qb, upg_w1, upg_w2, upg_b2, gate_norm_w, up_proj_w, pb,
           down_norm_w, pos_emb, db, down_proj_w, res_norm_w, cb, sb):
    x2d = x[0]
    gathered = _block(x2d, qb)
    up, mt = _route(gathered, upg_w1, upg_w2, upg_b2, gate_norm_w, up_proj_w)
    up = _block(up, pb)
    scattered = _scatter(mt, up, down_norm_w, pos_emb)
    pds = _block(scattered, db)
    pds = _mm(pds, [down_proj_w])
    query = _addln(x2d, pos_emb, res_norm_w)
    xout = _combine(pds, query, cb)
    xout = _block(xout, sb)
    return (xout[None], jnp.float32(0.0))
